# baseline (speedup 1.0000x reference)
import jax
import jax.numpy as jnp
from jax.experimental import pallas as pl
from jax.experimental.pallas import tpu as pltpu

_M = 4
_N = 8
_K = 2 * 768 * 2048
_CK = 131072
_T = _K // _CK


def _gate_body(x_ref, w_ref, b_ref, o_ref, acc_ref):
    @pl.when(pl.program_id(0) == 0)
    def _init():
        acc_ref[...] = jnp.zeros_like(acc_ref)

    acc_ref[...] += jax.lax.dot_general(
        x_ref[...], w_ref[...],
        dimension_numbers=(((1,), (1,)), ((), ())),
        preferred_element_type=jnp.float32,
    )

    @pl.when(pl.program_id(0) == _T - 1)
    def _finish():
        v = acc_ref[...] + b_ref[...]
        lane = jax.lax.broadcasted_iota(jnp.int32, v.shape, 1)

        def drop_one_max(u):
            m = jnp.max(u, axis=1, keepdims=True)
            first = jnp.min(
                jnp.where(u == m, lane, v.shape[1]), axis=1, keepdims=True
            )
            return jnp.where(lane == first, -jnp.inf, u)

        thr = jnp.max(drop_one_max(drop_one_max(v)), axis=1, keepdims=True)
        m1 = jnp.max(v, axis=1, keepdims=True)
        e = jnp.where(v > thr, jnp.exp(v - m1), 0.0)
        o_ref[...] = e / jnp.sum(e, axis=1, keepdims=True)


@jax.jit
def kernel(x, W, b):
    xf = x.reshape(x.shape[0], -1)
    return pl.pallas_call(
        _gate_body,
        grid=(_T,),
        in_specs=[
            pl.BlockSpec((_M, _CK), lambda i: (0, i)),
            pl.BlockSpec((_N, _CK), lambda i: (0, i)),
            pl.BlockSpec((1, _N), lambda i: (0, 0)),
        ],
        out_specs=pl.BlockSpec((_M, _N), lambda i: (0, 0)),
        out_shape=jax.ShapeDtypeStruct((_M, _N), jnp.float32),
        scratch_shapes=[pltpu.VMEM((_M, _N), jnp.float32)],
        compiler_params=pltpu.CompilerParams(
            dimension_semantics=("arbitrary",),
        ),
    )(xf, W, b.reshape(1, -1))
